# Initial kernel scaffold; baseline (speedup 1.0000x reference)
import jax
import jax.numpy as jnp
from jax.experimental import pallas as pl


def kernel(x, edge_index, edge_attr, layer, W_ne1, W_ne2, W_ee1, W_ee2, W_r1, b_r1, W_r2, b_r2, W_o1, b_o1, W_o2, b_o2, W_b1, b_b1, W_b2, b_b2, W_b3, b_b3, W_c1, b_c1, W_c2, b_c2, W_c3, b_c3, latent_norm):
    raise NotImplementedError("write your pallas kernel here")


# speedup vs baseline: 3.9275x; 3.9275x over previous
import functools

import jax
import jax.numpy as jnp
from jax import lax
from jax.experimental import pallas as pl
from jax.experimental.pallas import tpu as pltpu
from jax.experimental.pallas import tpu_sc as plsc

NC, NS = 2, 16
NW = NC * NS
P = 8
STCH = 2000

_relu = lambda v: jnp.maximum(v, 0.0)
_SC_PARAMS = pltpu.CompilerParams(use_tc_tiling_on_sc=False)


def _dot(a, b):
    return jnp.dot(a, b, preferred_element_type=jnp.float32)


def _node_enc_body(x_ref, w1_ref, w2_ref, o_ref):
    t = _relu(_dot(x_ref[...], w1_ref[...]))
    o_ref[...] = _relu(_dot(t, w2_ref[...]))


def _edge_body(g_ref, ea_ref, wg_ref, we1_ref, we2_ref, wc_ref, wr2_ref,
               br1_ref, br2_ref, o_ref):
    t1 = _relu(_dot(ea_ref[...], we1_ref[...]))
    e = _relu(_dot(t1, we2_ref[...]))
    c = _dot(e, wc_ref[...])
    hid = _relu(_dot(g_ref[...], wg_ref[...]) + c + br1_ref[...])
    o_ref[...] = _dot(hid, wr2_ref[...]) + br2_ref[...]


def _node_out_body(h_ref, p0_ref, p1_ref, wo1h_ref, wo1a_ref, bo1_ref,
                   wo2_ref, bo2_ref, wbc1_ref, bbc1_ref, w2_ref, b2_ref,
                   w3_ref, b3_ref, ln_ref, o_ref):
    agg = p0_ref[...] + p1_ref[...]
    hid = _relu(_dot(h_ref[...], wo1h_ref[...]) + _dot(agg, wo1a_ref[...])
                + bo1_ref[...])
    hn = _dot(hid, wo2_ref[...]) + bo2_ref[...]
    t = _relu(_dot(hn, wbc1_ref[...]) + bbc1_ref[...])
    t2 = _relu(_dot(t, w2_ref[...]) + b2_ref[...])
    o3 = _dot(t2, w3_ref[...]) + b3_ref[...]
    eps = 1e-6
    beta = eps + (1.0 - 2.0 * eps) * jax.nn.sigmoid(o3[:, 0:1])
    hc = o3[:, 1:3] * ln_ref[0, 0]
    pad = jnp.zeros((o3.shape[0], 5), jnp.float32)
    o_ref[...] = jnp.concatenate([beta, hc, pad], axis=1)


def _stage_loop(sid, n_st, body_fn):
    per_tile = (n_st + NS - 1) // NS

    def body(i, c):
        ch = sid + NS * i

        @pl.when(ch < n_st)
        def _():
            body_fn(ch)

        return c

    lax.fori_loop(0, per_tile, body, 0)


def _make_gather(n_idx, n_nodes, c2):
    epw = n_idx // NW
    n_st = n_nodes // STCH

    @functools.partial(
        pl.kernel,
        out_type=jax.ShapeDtypeStruct((n_idx, 8), jnp.float32),
        mesh=plsc.VectorSubcoreMesh(core_axis_name="c", subcore_axis_name="s"),
        compiler_params=_SC_PARAMS,
        scratch_types=[
            pltpu.VMEM((c2,), jnp.int32),
            pltpu.VMEM((c2, 8), jnp.float32),
            pltpu.VMEM((STCH, 8), jnp.float32),
            pltpu.VMEM_SHARED((n_nodes, 8), jnp.float32),
            pltpu.SemaphoreType.DMA,
        ],
    )
    def gather_k(h_hbm, idx_hbm, out_hbm, idx_v, rows_v, zbuf, table, sem):
        cid = lax.axis_index("c")
        sid = lax.axis_index("s")

        def stage(ch):
            pltpu.sync_copy(h_hbm.at[pl.ds(ch * STCH, STCH)], zbuf)
            pltpu.sync_copy(zbuf, table.at[pl.ds(ch * STCH, STCH)])

        _stage_loop(sid, n_st, stage)
        plsc.subcore_barrier()

        base = (sid * NC + cid) * epw

        def body(i, c):
            off = base + i * c2
            pltpu.sync_copy(idx_hbm.at[pl.ds(off, c2)], idx_v)
            pltpu.async_copy(table.at[idx_v], rows_v, sem).wait()
            pltpu.sync_copy(rows_v, out_hbm.at[pl.ds(off, c2)])
            return c

        lax.fori_loop(0, epw // c2, body, 0)

    return gather_k


def _make_scatter(n_nodes, n_edges, batch, kin):
    rows_pw = (n_edges // batch) // NW
    n_outer = rows_pw // kin
    n_st = n_nodes // STCH

    @functools.partial(
        pl.kernel,
        out_type=jax.ShapeDtypeStruct((NC, n_nodes, 8), jnp.float32),
        mesh=plsc.VectorSubcoreMesh(core_axis_name="c", subcore_axis_name="s"),
        compiler_params=_SC_PARAMS,
        scratch_types=[
            pltpu.VMEM((kin, batch), jnp.int32),
            pltpu.VMEM((kin * batch, 8), jnp.float32),
            pltpu.VMEM((STCH, 8), jnp.float32),
            pltpu.VMEM_SHARED((n_nodes, 8), jnp.float32),
        ],
    )
    def scatter_k(enew_hbm, dst2_hbm, zeros_hbm, out_hbm, idx_v, ebuf, zbuf,
                  shared):
        cid = lax.axis_index("c")
        sid = lax.axis_index("s")

        def zstage(ch):
            pltpu.sync_copy(zeros_hbm.at[pl.ds(ch * STCH, STCH)], zbuf)
            pltpu.sync_copy(zbuf, shared.at[pl.ds(ch * STCH, STCH)])

        _stage_loop(sid, n_st, zstage)
        plsc.subcore_barrier()

        row0 = (cid * NS + sid) * rows_pw

        def outer(i, c):
            rbase = row0 + i * kin
            pltpu.sync_copy(dst2_hbm.at[pl.ds(rbase, kin)], idx_v)
            pltpu.sync_copy(enew_hbm.at[pl.ds(rbase * batch, kin * batch)],
                            ebuf)
            for j in range(kin):
                pltpu.sync_copy(ebuf.at[pl.ds(j * batch, batch)],
                                shared.at[idx_v.at[j]], add=True)
            return c

        lax.fori_loop(0, n_outer, outer, 0)
        plsc.subcore_barrier()

        def drain(ch):
            pltpu.sync_copy(shared.at[pl.ds(ch * STCH, STCH)], zbuf)
            pltpu.sync_copy(zbuf, out_hbm.at[cid, pl.ds(ch * STCH, STCH)])

        _stage_loop(sid, n_st, drain)

    return scatter_k


def kernel(x, edge_index, edge_attr, layer,
           W_ne1, W_ne2, W_ee1, W_ee2,
           W_r1, b_r1, W_r2, b_r2, W_o1, b_o1, W_o2, b_o2,
           W_b1, b_b1, W_b2, b_b2, W_b3, b_b3,
           W_c1, b_c1, W_c2, b_c2, W_c3, b_c3,
           latent_norm):
    f32 = jnp.float32
    N, D = x.shape
    E = edge_attr.shape[0]
    HID = W_ne1.shape[1]

    eye = jnp.eye(P, dtype=f32)
    w_ne2p = jnp.zeros((HID, 8), f32).at[:, :5].set(W_ne2)
    w1sd = (jnp.zeros((16, HID), f32)
            .at[0:5].set(W_r1[0:5]).at[8:13].set(W_r1[5:10]))
    wg = jnp.kron(eye, w1sd)
    we1 = jnp.kron(eye, W_ee1)
    we2 = jnp.kron(eye, W_ee2)
    wc = jnp.kron(eye, W_r1[10:14])
    wr2 = jnp.kron(eye, jnp.zeros((HID, 8), f32).at[:, :4].set(W_r2))
    br1 = jnp.tile(b_r1, P).reshape(1, P * HID)
    br2 = jnp.tile(jnp.zeros((8,), f32).at[:4].set(b_r2), P).reshape(1, 8 * P)

    wo1h = jnp.zeros((8, HID), f32).at[0:5].set(W_o1[0:5])
    wo1a = jnp.zeros((8, HID), f32).at[0:4].set(W_o1[5:9])
    bo1 = b_o1.reshape(1, HID)
    wo2 = jnp.zeros((HID, 8), f32).at[:, :5].set(W_o2)
    bo2 = jnp.zeros((1, 8), f32).at[0, :5].set(b_o2)
    wbc1 = jnp.zeros((8, 2 * HID), f32).at[0:5].set(
        jnp.concatenate([W_b1, W_c1], axis=1))
    bbc1 = jnp.concatenate([b_b1, b_c1]).reshape(1, 2 * HID)
    w2 = (jnp.zeros((2 * HID, 2 * HID), f32)
          .at[:HID, :HID].set(W_b2).at[HID:, HID:].set(W_c2))
    b2 = jnp.concatenate([b_b2, b_c2]).reshape(1, 2 * HID)
    w3 = (jnp.zeros((2 * HID, 8), f32)
          .at[:HID, 0:1].set(W_b3).at[HID:, 1:3].set(W_c3))
    b3 = jnp.concatenate([b_b3, b_c3, jnp.zeros((5,), f32)]).reshape(1, 8)
    ln = latent_norm.reshape(1, 1)

    idx2 = jnp.stack([edge_index[0], edge_index[1]], axis=1).reshape(2 * E)

    BN = 2000
    h8 = pl.pallas_call(
        _node_enc_body,
        grid=(N // BN,),
        in_specs=[
            pl.BlockSpec((BN, D), lambda i: (i, 0)),
            pl.BlockSpec((D, HID), lambda i: (0, 0)),
            pl.BlockSpec((HID, 8), lambda i: (0, 0)),
        ],
        out_specs=pl.BlockSpec((BN, 8), lambda i: (i, 0)),
        out_shape=jax.ShapeDtypeStruct((N, 8), f32),
    )(x, W_ne1, w_ne2p)

    g = _make_gather(2 * E, N, 2000)(h8, idx2)

    R = E // P
    BR = 2000
    gp = g.reshape(R, 16 * P)
    eap = edge_attr.reshape(R, 4 * P)
    enew = pl.pallas_call(
        _edge_body,
        grid=(R // BR,),
        in_specs=[
            pl.BlockSpec((BR, 16 * P), lambda i: (i, 0)),
            pl.BlockSpec((BR, 4 * P), lambda i: (i, 0)),
            pl.BlockSpec((16 * P, HID * P), lambda i: (0, 0)),
            pl.BlockSpec((4 * P, HID * P), lambda i: (0, 0)),
            pl.BlockSpec((HID * P, 4 * P), lambda i: (0, 0)),
            pl.BlockSpec((4 * P, HID * P), lambda i: (0, 0)),
            pl.BlockSpec((HID * P, 8 * P), lambda i: (0, 0)),
            pl.BlockSpec((1, HID * P), lambda i: (0, 0)),
            pl.BlockSpec((1, 8 * P), lambda i: (0, 0)),
        ],
        out_specs=pl.BlockSpec((BR, 8 * P), lambda i: (i, 0)),
        out_shape=jax.ShapeDtypeStruct((R, 8 * P), f32),
    )(gp, eap, wg, we1, we2, wc, wr2, br1, br2)

    enew8 = enew.reshape(E, 8)
    BATCH, KIN = 80, 10
    dst2 = edge_index[1].reshape(E // BATCH, BATCH)
    partials = _make_scatter(N, E, BATCH, KIN)(
        enew8, dst2, jnp.zeros((N, 8), f32))

    out8 = pl.pallas_call(
        _node_out_body,
        grid=(N // BN,),
        in_specs=[
            pl.BlockSpec((BN, 8), lambda i: (i, 0)),
            pl.BlockSpec((BN, 8), lambda i: (i, 0)),
            pl.BlockSpec((BN, 8), lambda i: (i, 0)),
            pl.BlockSpec((8, HID), lambda i: (0, 0)),
            pl.BlockSpec((8, HID), lambda i: (0, 0)),
            pl.BlockSpec((1, HID), lambda i: (0, 0)),
            pl.BlockSpec((HID, 8), lambda i: (0, 0)),
            pl.BlockSpec((1, 8), lambda i: (0, 0)),
            pl.BlockSpec((8, 2 * HID), lambda i: (0, 0)),
            pl.BlockSpec((1, 2 * HID), lambda i: (0, 0)),
            pl.BlockSpec((2 * HID, 2 * HID), lambda i: (0, 0)),
            pl.BlockSpec((1, 2 * HID), lambda i: (0, 0)),
            pl.BlockSpec((2 * HID, 8), lambda i: (0, 0)),
            pl.BlockSpec((1, 8), lambda i: (0, 0)),
            pl.BlockSpec((1, 1), lambda i: (0, 0)),
        ],
        out_specs=pl.BlockSpec((BN, 8), lambda i: (i, 0)),
        out_shape=jax.ShapeDtypeStruct((N, 8), f32),
    )(h8, partials[0], partials[1], wo1h, wo1a, bo1, wo2, bo2,
      wbc1, bbc1, w2, b2, w3, b3, ln)

    return out8[:, 1:3], out8[:, 0]


# speedup vs baseline: 4.9194x; 1.2525x over previous
import functools

import jax
import jax.numpy as jnp
from jax import lax
from jax.experimental import pallas as pl
from jax.experimental.pallas import tpu as pltpu
from jax.experimental.pallas import tpu_sc as plsc

NC, NS = 2, 16
NW = NC * NS
P = 8
STCH = 2000

_relu = lambda v: jnp.maximum(v, 0.0)
_SC_PARAMS = pltpu.CompilerParams(use_tc_tiling_on_sc=False)


def _dot(a, b):
    return jnp.dot(a, b, preferred_element_type=jnp.float32)


def _node_enc_body(x_ref, w1_ref, w2_ref, o_ref):
    t = _relu(_dot(x_ref[...], w1_ref[...]))
    o_ref[...] = _relu(_dot(t, w2_ref[...]))


def _edge_body(g_ref, ea_ref, wg_ref, we1_ref, we2_ref, wc_ref, wr2_ref,
               br1_ref, br2_ref, o_ref):
    br = g_ref.shape[0]
    ea = ea_ref[...].reshape(4, br, P).transpose(1, 2, 0).reshape(br, 4 * P)
    t1 = _relu(_dot(ea, we1_ref[...]))
    e = _relu(_dot(t1, we2_ref[...]))
    c = _dot(e, wc_ref[...])
    hid = _relu(_dot(g_ref[...], wg_ref[...]) + c + br1_ref[...])
    o_ref[...] = _dot(hid, wr2_ref[...]) + br2_ref[...]


def _node_out_body(h_ref, p0_ref, p1_ref, wo1h_ref, wo1a_ref, bo1_ref,
                   wo2_ref, bo2_ref, wbc1_ref, bbc1_ref, w2_ref, b2_ref,
                   w3_ref, b3_ref, ln_ref, o_ref):
    agg = p0_ref[...] + p1_ref[...]
    hid = _relu(_dot(h_ref[...], wo1h_ref[...]) + _dot(agg, wo1a_ref[...])
                + bo1_ref[...])
    hn = _dot(hid, wo2_ref[...]) + bo2_ref[...]
    t = _relu(_dot(hn, wbc1_ref[...]) + bbc1_ref[...])
    t2 = _relu(_dot(t, w2_ref[...]) + b2_ref[...])
    o3 = _dot(t2, w3_ref[...]) + b3_ref[...]
    eps = 1e-6
    beta = eps + (1.0 - 2.0 * eps) * jax.nn.sigmoid(o3[:, 0:1])
    hc = o3[:, 1:3] * ln_ref[0, 0]
    pad = jnp.zeros((o3.shape[0], 5), jnp.float32)
    o_ref[...] = jnp.concatenate([beta, hc, pad], axis=1)


def _stage_loop(sid, n_st, body_fn):
    per_tile = (n_st + NS - 1) // NS

    def body(i, c):
        ch = sid + NS * i

        @pl.when(ch < n_st)
        def _():
            body_fn(ch)

        return c

    lax.fori_loop(0, per_tile, body, 0)


def _make_gather(n_idx, n_nodes, c2):
    epw = n_idx // NW
    n_st = n_nodes // STCH

    @functools.partial(
        pl.kernel,
        out_type=jax.ShapeDtypeStruct((n_idx, 8), jnp.float32),
        mesh=plsc.VectorSubcoreMesh(core_axis_name="c", subcore_axis_name="s"),
        compiler_params=_SC_PARAMS,
        scratch_types=[
            pltpu.VMEM((c2,), jnp.int32),
            pltpu.VMEM((c2, 8), jnp.float32),
            pltpu.VMEM((STCH, 8), jnp.float32),
            pltpu.VMEM_SHARED((n_nodes, 8), jnp.float32),
            pltpu.SemaphoreType.DMA,
        ],
    )
    def gather_k(h_hbm, idx_hbm, out_hbm, idx_v, rows_v, zbuf, table, sem):
        cid = lax.axis_index("c")
        sid = lax.axis_index("s")

        def stage(ch):
            pltpu.sync_copy(h_hbm.at[pl.ds(ch * STCH, STCH)], zbuf)
            pltpu.sync_copy(zbuf, table.at[pl.ds(ch * STCH, STCH)])

        _stage_loop(sid, n_st, stage)
        plsc.subcore_barrier()

        base = (sid * NC + cid) * epw

        def body(i, c):
            off = base + i * c2
            pltpu.sync_copy(idx_hbm.at[pl.ds(off, c2)], idx_v)
            pltpu.async_copy(table.at[idx_v], rows_v, sem).wait()
            pltpu.sync_copy(rows_v, out_hbm.at[pl.ds(off, c2)])
            return c

        lax.fori_loop(0, epw // c2, body, 0)

    return gather_k


def _make_scatter(n_nodes, n_edges, batch, kin):
    rows_pw = (n_edges // batch) // NW
    n_outer = rows_pw // kin
    n_st = n_nodes // STCH

    @functools.partial(
        pl.kernel,
        out_type=jax.ShapeDtypeStruct((NC, n_nodes, 8), jnp.float32),
        mesh=plsc.VectorSubcoreMesh(core_axis_name="c", subcore_axis_name="s"),
        compiler_params=_SC_PARAMS,
        scratch_types=[
            pltpu.VMEM((kin, batch), jnp.int32),
            pltpu.VMEM((kin * batch, 8), jnp.float32),
            pltpu.VMEM((STCH, 8), jnp.float32),
            pltpu.VMEM_SHARED((n_nodes, 8), jnp.float32),
        ],
    )
    def scatter_k(enew_hbm, dst2_hbm, zeros_hbm, out_hbm, idx_v, ebuf, zbuf,
                  shared):
        cid = lax.axis_index("c")
        sid = lax.axis_index("s")

        def zstage(ch):
            pltpu.sync_copy(zeros_hbm.at[pl.ds(ch * STCH, STCH)], zbuf)
            pltpu.sync_copy(zbuf, shared.at[pl.ds(ch * STCH, STCH)])

        _stage_loop(sid, n_st, zstage)
        plsc.subcore_barrier()

        row0 = (cid * NS + sid) * rows_pw

        def outer(i, c):
            rbase = row0 + i * kin
            pltpu.sync_copy(dst2_hbm.at[pl.ds(rbase, kin)], idx_v)
            pltpu.sync_copy(enew_hbm.at[pl.ds(rbase * batch, kin * batch)],
                            ebuf)
            for j in range(kin):
                pltpu.sync_copy(ebuf.at[pl.ds(j * batch, batch)],
                                shared.at[idx_v.at[j]], add=True)
            return c

        lax.fori_loop(0, n_outer, outer, 0)
        plsc.subcore_barrier()

        def drain(ch):
            pltpu.sync_copy(shared.at[pl.ds(ch * STCH, STCH)], zbuf)
            pltpu.sync_copy(zbuf, out_hbm.at[cid, pl.ds(ch * STCH, STCH)])

        _stage_loop(sid, n_st, drain)

    return scatter_k


def kernel(x, edge_index, edge_attr, layer,
           W_ne1, W_ne2, W_ee1, W_ee2,
           W_r1, b_r1, W_r2, b_r2, W_o1, b_o1, W_o2, b_o2,
           W_b1, b_b1, W_b2, b_b2, W_b3, b_b3,
           W_c1, b_c1, W_c2, b_c2, W_c3, b_c3,
           latent_norm):
    f32 = jnp.float32
    N, D = x.shape
    E = edge_attr.shape[0]
    HID = W_ne1.shape[1]

    eye = jnp.eye(P, dtype=f32)
    w_ne2p = jnp.zeros((HID, 8), f32).at[:, :5].set(W_ne2)
    w1sd = (jnp.zeros((16, HID), f32)
            .at[0:5].set(W_r1[0:5]).at[8:13].set(W_r1[5:10]))
    wg = jnp.kron(eye, w1sd)
    we1 = jnp.kron(eye, W_ee1)
    we2 = jnp.kron(eye, W_ee2)
    wc = jnp.kron(eye, W_r1[10:14])
    wr2 = jnp.kron(eye, jnp.zeros((HID, 8), f32).at[:, :4].set(W_r2))
    br1 = jnp.tile(b_r1, P).reshape(1, P * HID)
    br2 = jnp.tile(jnp.zeros((8,), f32).at[:4].set(b_r2), P).reshape(1, 8 * P)

    wo1h = jnp.zeros((8, HID), f32).at[0:5].set(W_o1[0:5])
    wo1a = jnp.zeros((8, HID), f32).at[0:4].set(W_o1[5:9])
    bo1 = b_o1.reshape(1, HID)
    wo2 = jnp.zeros((HID, 8), f32).at[:, :5].set(W_o2)
    bo2 = jnp.zeros((1, 8), f32).at[0, :5].set(b_o2)
    wbc1 = jnp.zeros((8, 2 * HID), f32).at[0:5].set(
        jnp.concatenate([W_b1, W_c1], axis=1))
    bbc1 = jnp.concatenate([b_b1, b_c1]).reshape(1, 2 * HID)
    w2 = (jnp.zeros((2 * HID, 2 * HID), f32)
          .at[:HID, :HID].set(W_b2).at[HID:, HID:].set(W_c2))
    b2 = jnp.concatenate([b_b2, b_c2]).reshape(1, 2 * HID)
    w3 = (jnp.zeros((2 * HID, 8), f32)
          .at[:HID, 0:1].set(W_b3).at[HID:, 1:3].set(W_c3))
    b3 = jnp.concatenate([b_b3, b_c3, jnp.zeros((5,), f32)]).reshape(1, 8)
    ln = latent_norm.reshape(1, 1)

    idx2 = jnp.stack([edge_index[0], edge_index[1]], axis=1).reshape(2 * E)

    BN = 2000
    h8 = pl.pallas_call(
        _node_enc_body,
        grid=(N // BN,),
        in_specs=[
            pl.BlockSpec((BN, D), lambda i: (i, 0)),
            pl.BlockSpec((D, HID), lambda i: (0, 0)),
            pl.BlockSpec((HID, 8), lambda i: (0, 0)),
        ],
        out_specs=pl.BlockSpec((BN, 8), lambda i: (i, 0)),
        out_shape=jax.ShapeDtypeStruct((N, 8), f32),
    )(x, W_ne1, w_ne2p)

    g = _make_gather(2 * E, N, 2000)(h8, idx2)

    R = E // P
    BR = 2000
    gp = g.reshape(R, 16 * P)
    eat = edge_attr.T
    enew = pl.pallas_call(
        _edge_body,
        grid=(R // BR,),
        in_specs=[
            pl.BlockSpec((BR, 16 * P), lambda i: (i, 0)),
            pl.BlockSpec((4, BR * P), lambda i: (0, i)),
            pl.BlockSpec((16 * P, HID * P), lambda i: (0, 0)),
            pl.BlockSpec((4 * P, HID * P), lambda i: (0, 0)),
            pl.BlockSpec((HID * P, 4 * P), lambda i: (0, 0)),
            pl.BlockSpec((4 * P, HID * P), lambda i: (0, 0)),
            pl.BlockSpec((HID * P, 8 * P), lambda i: (0, 0)),
            pl.BlockSpec((1, HID * P), lambda i: (0, 0)),
            pl.BlockSpec((1, 8 * P), lambda i: (0, 0)),
        ],
        out_specs=pl.BlockSpec((BR, 8 * P), lambda i: (i, 0)),
        out_shape=jax.ShapeDtypeStruct((R, 8 * P), f32),
    )(gp, eat, wg, we1, we2, wc, wr2, br1, br2)

    enew8 = enew.reshape(E, 8)
    BATCH, KIN = 80, 10
    dst2 = edge_index[1].reshape(E // BATCH, BATCH)
    partials = _make_scatter(N, E, BATCH, KIN)(
        enew8, dst2, jnp.zeros((N, 8), f32))

    out8 = pl.pallas_call(
        _node_out_body,
        grid=(N // BN,),
        in_specs=[
            pl.BlockSpec((BN, 8), lambda i: (i, 0)),
            pl.BlockSpec((BN, 8), lambda i: (i, 0)),
            pl.BlockSpec((BN, 8), lambda i: (i, 0)),
            pl.BlockSpec((8, HID), lambda i: (0, 0)),
            pl.BlockSpec((8, HID), lambda i: (0, 0)),
            pl.BlockSpec((1, HID), lambda i: (0, 0)),
            pl.BlockSpec((HID, 8), lambda i: (0, 0)),
            pl.BlockSpec((1, 8), lambda i: (0, 0)),
            pl.BlockSpec((8, 2 * HID), lambda i: (0, 0)),
            pl.BlockSpec((1, 2 * HID), lambda i: (0, 0)),
            pl.BlockSpec((2 * HID, 2 * HID), lambda i: (0, 0)),
            pl.BlockSpec((1, 2 * HID), lambda i: (0, 0)),
            pl.BlockSpec((2 * HID, 8), lambda i: (0, 0)),
            pl.BlockSpec((1, 8), lambda i: (0, 0)),
            pl.BlockSpec((1, 1), lambda i: (0, 0)),
        ],
        out_specs=pl.BlockSpec((BN, 8), lambda i: (i, 0)),
        out_shape=jax.ShapeDtypeStruct((N, 8), f32),
    )(h8, partials[0], partials[1], wo1h, wo1a, bo1, wo2, bo2,
      wbc1, bbc1, w2, b2, w3, b3, ln)

    return out8[:, 1:3], out8[:, 0]


# speedup vs baseline: 11.1592x; 2.2684x over previous
import functools

import jax
import jax.numpy as jnp
from jax import lax
from jax.experimental import pallas as pl
from jax.experimental.pallas import tpu as pltpu
from jax.experimental.pallas import tpu_sc as plsc

NC, NS = 2, 16
NW = NC * NS
P = 8
STCH = 2000

_relu = lambda v: jnp.maximum(v, 0.0)
_SC_PARAMS = pltpu.CompilerParams(use_tc_tiling_on_sc=False)


def _dot(a, b):
    return jnp.dot(a, b, preferred_element_type=jnp.float32)


def _node_enc_body(x_ref, w1_ref, w2_ref, o_ref):
    t = _relu(_dot(x_ref[...], w1_ref[...]))
    o_ref[...] = _relu(_dot(t, w2_ref[...]))


def _edge_body(g_ref, ea_ref, wg_ref, we1_ref, we2_ref, wc_ref, wr2_ref,
               br1_ref, br2_ref, o_ref):
    br = g_ref.shape[0]
    ea = ea_ref[...].reshape(4, br, P).transpose(1, 2, 0).reshape(br, 4 * P)
    t1 = _relu(_dot(ea, we1_ref[...]))
    e = _relu(_dot(t1, we2_ref[...]))
    c = _dot(e, wc_ref[...])
    hid = _relu(_dot(g_ref[...], wg_ref[...]) + c + br1_ref[...])
    o_ref[...] = _dot(hid, wr2_ref[...]) + br2_ref[...]


def _node_out_body(h_ref, p0_ref, p1_ref, wo1h_ref, wo1a_ref, bo1_ref,
                   wo2_ref, bo2_ref, wbc1_ref, bbc1_ref, w2_ref, b2_ref,
                   w3_ref, b3_ref, ln_ref, o_ref):
    agg = p0_ref[...] + p1_ref[...]
    hid = _relu(_dot(h_ref[...], wo1h_ref[...]) + _dot(agg, wo1a_ref[...])
                + bo1_ref[...])
    hn = _dot(hid, wo2_ref[...]) + bo2_ref[...]
    t = _relu(_dot(hn, wbc1_ref[...]) + bbc1_ref[...])
    t2 = _relu(_dot(t, w2_ref[...]) + b2_ref[...])
    o3 = _dot(t2, w3_ref[...]) + b3_ref[...]
    eps = 1e-6
    beta = eps + (1.0 - 2.0 * eps) * jax.nn.sigmoid(o3[:, 0:1])
    hc = o3[:, 1:3] * ln_ref[0, 0]
    pad = jnp.zeros((o3.shape[0], 5), jnp.float32)
    o_ref[...] = jnp.concatenate([beta, hc, pad], axis=1)


def _stage_loop(sid, n_st, body_fn):
    per_tile = (n_st + NS - 1) // NS

    def body(i, c):
        ch = sid + NS * i

        @pl.when(ch < n_st)
        def _():
            body_fn(ch)

        return c

    lax.fori_loop(0, per_tile, body, 0)


def _make_gather(n_edges, n_nodes):
    n_ch = n_edges // 1024
    base_n = n_ch // NW
    rem = n_ch - base_n * NW
    n_st = n_nodes // STCH

    @functools.partial(
        pl.kernel,
        out_type=jax.ShapeDtypeStruct((n_edges, 16), jnp.float32),
        mesh=plsc.VectorSubcoreMesh(core_axis_name="c", subcore_axis_name="s"),
        compiler_params=_SC_PARAMS,
        scratch_types=[
            pltpu.VMEM((8, 2, 128), jnp.int32),
            pltpu.VMEM((1024, 8), jnp.float32),
            pltpu.VMEM((1024, 8), jnp.float32),
            pltpu.VMEM((STCH, 8), jnp.float32),
            pltpu.VMEM_SHARED((n_nodes, 8), jnp.float32),
            pltpu.SemaphoreType.DMA,
        ],
    )
    def gather_k(h_hbm, idx3_hbm, out_hbm, idx_v, gs_v, gd_v, zbuf, table,
                 sem):
        cid = lax.axis_index("c")
        sid = lax.axis_index("s")

        def stage(ch):
            pltpu.sync_copy(h_hbm.at[pl.ds(ch * STCH, STCH)], zbuf)
            pltpu.sync_copy(zbuf, table.at[pl.ds(ch * STCH, STCH)])

        _stage_loop(sid, n_st, stage)
        plsc.subcore_barrier()

        w = sid * NC + cid
        ch0 = base_n * w + jnp.minimum(w, rem)

        def body(i, c):
            ch = ch0 + i
            pltpu.sync_copy(idx3_hbm.at[pl.ds(ch * 8, 8)], idx_v)
            for t in range(8):
                pltpu.async_copy(table.at[idx_v.at[t, 0]],
                                 gs_v.at[pl.ds(t * 128, 128)], sem).wait()
                pltpu.async_copy(table.at[idx_v.at[t, 1]],
                                 gd_v.at[pl.ds(t * 128, 128)], sem).wait()
            pltpu.sync_copy(gs_v,
                            out_hbm.at[pl.ds(ch * 1024, 1024), pl.ds(0, 8)])
            pltpu.sync_copy(gd_v,
                            out_hbm.at[pl.ds(ch * 1024, 1024), pl.ds(8, 8)])
            return c

        lax.fori_loop(0, base_n + jnp.where(w < rem, 1, 0), body, 0)

    return gather_k


def _make_scatter(n_nodes, n_edges):
    n_ch = n_edges // 1024
    base_n = n_ch // NW
    rem = n_ch - base_n * NW
    n_st = n_nodes // STCH

    @functools.partial(
        pl.kernel,
        out_type=jax.ShapeDtypeStruct((NC, n_nodes, 8), jnp.float32),
        mesh=plsc.VectorSubcoreMesh(core_axis_name="c", subcore_axis_name="s"),
        compiler_params=_SC_PARAMS,
        scratch_types=[
            pltpu.VMEM((8, 2, 128), jnp.int32),
            pltpu.VMEM((1024, 8), jnp.float32),
            pltpu.VMEM((STCH, 8), jnp.float32),
            pltpu.VMEM_SHARED((n_nodes, 8), jnp.float32),
        ],
    )
    def scatter_k(enew_hbm, idx3_hbm, zeros_hbm, out_hbm, idx_v, ebuf, zbuf,
                  shared):
        cid = lax.axis_index("c")
        sid = lax.axis_index("s")

        def zstage(ch):
            pltpu.sync_copy(zeros_hbm.at[pl.ds(ch * STCH, STCH)], zbuf)
            pltpu.sync_copy(zbuf, shared.at[pl.ds(ch * STCH, STCH)])

        _stage_loop(sid, n_st, zstage)
        plsc.subcore_barrier()

        w = cid * NS + sid
        ch0 = base_n * w + jnp.minimum(w, rem)

        def outer(i, c):
            ch = ch0 + i
            pltpu.sync_copy(idx3_hbm.at[pl.ds(ch * 8, 8)], idx_v)
            pltpu.sync_copy(enew_hbm.at[pl.ds(ch * 1024, 1024)], ebuf)
            for t in range(8):
                pltpu.sync_copy(ebuf.at[pl.ds(t * 128, 128)],
                                shared.at[idx_v.at[t, 1]], add=True)
            return c

        lax.fori_loop(0, base_n + jnp.where(w < rem, 1, 0), outer, 0)
        plsc.subcore_barrier()

        def drain(ch):
            pltpu.sync_copy(shared.at[pl.ds(ch * STCH, STCH)], zbuf)
            pltpu.sync_copy(zbuf, out_hbm.at[cid, pl.ds(ch * STCH, STCH)])

        _stage_loop(sid, n_st, drain)

    return scatter_k


def kernel(x, edge_index, edge_attr, layer,
           W_ne1, W_ne2, W_ee1, W_ee2,
           W_r1, b_r1, W_r2, b_r2, W_o1, b_o1, W_o2, b_o2,
           W_b1, b_b1, W_b2, b_b2, W_b3, b_b3,
           W_c1, b_c1, W_c2, b_c2, W_c3, b_c3,
           latent_norm):
    f32 = jnp.float32
    N, D = x.shape
    E = edge_attr.shape[0]
    HID = W_ne1.shape[1]

    eye = jnp.eye(P, dtype=f32)
    w_ne2p = jnp.zeros((HID, 8), f32).at[:, :5].set(W_ne2)
    w1sd = (jnp.zeros((16, HID), f32)
            .at[0:5].set(W_r1[0:5]).at[8:13].set(W_r1[5:10]))
    wg = jnp.kron(eye, w1sd)
    we1 = jnp.kron(eye, W_ee1)
    we2 = jnp.kron(eye, W_ee2)
    wc = jnp.kron(eye, W_r1[10:14])
    wr2 = jnp.kron(eye, jnp.zeros((HID, 8), f32).at[:, :4].set(W_r2))
    br1 = jnp.tile(b_r1, P).reshape(1, P * HID)
    br2 = jnp.tile(jnp.zeros((8,), f32).at[:4].set(b_r2), P).reshape(1, 8 * P)

    wo1h = jnp.zeros((8, HID), f32).at[0:5].set(W_o1[0:5])
    wo1a = jnp.zeros((8, HID), f32).at[0:4].set(W_o1[5:9])
    bo1 = b_o1.reshape(1, HID)
    wo2 = jnp.zeros((HID, 8), f32).at[:, :5].set(W_o2)
    bo2 = jnp.zeros((1, 8), f32).at[0, :5].set(b_o2)
    wbc1 = jnp.zeros((8, 2 * HID), f32).at[0:5].set(
        jnp.concatenate([W_b1, W_c1], axis=1))
    bbc1 = jnp.concatenate([b_b1, b_c1]).reshape(1, 2 * HID)
    w2 = (jnp.zeros((2 * HID, 2 * HID), f32)
          .at[:HID, :HID].set(W_b2).at[HID:, HID:].set(W_c2))
    b2 = jnp.concatenate([b_b2, b_c2]).reshape(1, 2 * HID)
    w3 = (jnp.zeros((2 * HID, 8), f32)
          .at[:HID, 0:1].set(W_b3).at[HID:, 1:3].set(W_c3))
    b3 = jnp.concatenate([b_b3, b_c3, jnp.zeros((5,), f32)]).reshape(1, 8)
    ln = latent_norm.reshape(1, 1)

    idx3 = edge_index.reshape(2, E // 128, 128).transpose(1, 0, 2)

    BN = 2000
    h8 = pl.pallas_call(
        _node_enc_body,
        grid=(N // BN,),
        in_specs=[
            pl.BlockSpec((BN, D), lambda i: (i, 0)),
            pl.BlockSpec((D, HID), lambda i: (0, 0)),
            pl.BlockSpec((HID, 8), lambda i: (0, 0)),
        ],
        out_specs=pl.BlockSpec((BN, 8), lambda i: (i, 0)),
        out_shape=jax.ShapeDtypeStruct((N, 8), f32),
    )(x, W_ne1, w_ne2p)

    g = _make_gather(E, N)(h8, idx3)

    R = E // P
    BR = 2000
    gp = g.reshape(R, 16 * P)
    eat = edge_attr.T
    enew = pl.pallas_call(
        _edge_body,
        grid=(R // BR,),
        in_specs=[
            pl.BlockSpec((BR, 16 * P), lambda i: (i, 0)),
            pl.BlockSpec((4, BR * P), lambda i: (0, i)),
            pl.BlockSpec((16 * P, HID * P), lambda i: (0, 0)),
            pl.BlockSpec((4 * P, HID * P), lambda i: (0, 0)),
            pl.BlockSpec((HID * P, 4 * P), lambda i: (0, 0)),
            pl.BlockSpec((4 * P, HID * P), lambda i: (0, 0)),
            pl.BlockSpec((HID * P, 8 * P), lambda i: (0, 0)),
            pl.BlockSpec((1, HID * P), lambda i: (0, 0)),
            pl.BlockSpec((1, 8 * P), lambda i: (0, 0)),
        ],
        out_specs=pl.BlockSpec((BR, 8 * P), lambda i: (i, 0)),
        out_shape=jax.ShapeDtypeStruct((R, 8 * P), f32),
    )(gp, eat, wg, we1, we2, wc, wr2, br1, br2)

    enew8 = enew.reshape(E, 8)
    partials = _make_scatter(N, E)(enew8, idx3, jnp.zeros((N, 8), f32))

    out8 = pl.pallas_call(
        _node_out_body,
        grid=(N // BN,),
        in_specs=[
            pl.BlockSpec((BN, 8), lambda i: (i, 0)),
            pl.BlockSpec((BN, 8), lambda i: (i, 0)),
            pl.BlockSpec((BN, 8), lambda i: (i, 0)),
            pl.BlockSpec((8, HID), lambda i: (0, 0)),
            pl.BlockSpec((8, HID), lambda i: (0, 0)),
            pl.BlockSpec((1, HID), lambda i: (0, 0)),
            pl.BlockSpec((HID, 8), lambda i: (0, 0)),
            pl.BlockSpec((1, 8), lambda i: (0, 0)),
            pl.BlockSpec((8, 2 * HID), lambda i: (0, 0)),
            pl.BlockSpec((1, 2 * HID), lambda i: (0, 0)),
            pl.BlockSpec((2 * HID, 2 * HID), lambda i: (0, 0)),
            pl.BlockSpec((1, 2 * HID), lambda i: (0, 0)),
            pl.BlockSpec((2 * HID, 8), lambda i: (0, 0)),
            pl.BlockSpec((1, 8), lambda i: (0, 0)),
            pl.BlockSpec((1, 1), lambda i: (0, 0)),
        ],
        out_specs=pl.BlockSpec((BN, 8), lambda i: (i, 0)),
        out_shape=jax.ShapeDtypeStruct((N, 8), f32),
    )(h8, partials[0], partials[1], wo1h, wo1a, bo1, wo2, bo2,
      wbc1, bbc1, w2, b2, w3, b3, ln)

    return out8[:, 1:3], out8[:, 0]


# speedup vs baseline: 18.1031x; 1.6223x over previous
import functools

import jax
import jax.numpy as jnp
from jax import lax
from jax.experimental import pallas as pl
from jax.experimental.pallas import tpu as pltpu
from jax.experimental.pallas import tpu_sc as plsc

NC, NS = 2, 16
NW = NC * NS
P = 8
STCH = 2000

_relu = lambda v: jnp.maximum(v, 0.0)
_SC_PARAMS = pltpu.CompilerParams(use_tc_tiling_on_sc=False)
_SC_PARAMS_NL = pltpu.CompilerParams(use_tc_tiling_on_sc=False,
                                     needs_layout_passes=False)


def _dot(a, b):
    return jnp.dot(a, b, preferred_element_type=jnp.float32)


def _node_enc_body(x_ref, w1_ref, w2_ref, o_ref):
    t = _relu(_dot(x_ref[...], w1_ref[...]))
    o_ref[...] = _relu(_dot(t, w2_ref[...]))


def _edge_body(g_ref, ea_ref, wg_ref, we1_ref, we2_ref, wc_ref, wr2_ref,
               br1_ref, br2_ref, o_ref):
    t1 = _relu(_dot(ea_ref[...], we1_ref[...]))
    e = _relu(_dot(t1, we2_ref[...]))
    c = _dot(e, wc_ref[...])
    hid = _relu(_dot(g_ref[...], wg_ref[...]) + c + br1_ref[...])
    o_ref[...] = _dot(hid, wr2_ref[...]) + br2_ref[...]


def _node_out_body(h_ref, p0_ref, p1_ref, wo1h_ref, wo1a_ref, bo1_ref,
                   wo2_ref, bo2_ref, wbc1_ref, bbc1_ref, w2_ref, b2_ref,
                   w3_ref, b3_ref, ln_ref, o_ref):
    agg = p0_ref[...] + p1_ref[...]
    hid = _relu(_dot(h_ref[...], wo1h_ref[...]) + _dot(agg, wo1a_ref[...])
                + bo1_ref[...])
    hn = _dot(hid, wo2_ref[...]) + bo2_ref[...]
    t = _relu(_dot(hn, wbc1_ref[...]) + bbc1_ref[...])
    t2 = _relu(_dot(t, w2_ref[...]) + b2_ref[...])
    o3 = _dot(t2, w3_ref[...]) + b3_ref[...]
    eps = 1e-6
    beta = eps + (1.0 - 2.0 * eps) * jax.nn.sigmoid(o3[:, 0:1])
    hc = o3[:, 1:3] * ln_ref[0, 0]
    pad = jnp.zeros((o3.shape[0], 5), jnp.float32)
    o_ref[...] = jnp.concatenate([beta, hc, pad], axis=1)


def _stage_loop(sid, n_st, body_fn):
    per_tile = (n_st + NS - 1) // NS

    def body(i, c):
        ch = sid + NS * i

        @pl.when(ch < n_st)
        def _():
            body_fn(ch)

        return c

    lax.fori_loop(0, per_tile, body, 0)


def _make_gather(n_edges, n_nodes):
    n_ch = n_edges // 1024
    base_n = n_ch // NW
    rem = n_ch - base_n * NW
    n_st = n_nodes // STCH

    @functools.partial(
        pl.kernel,
        out_type=[
            jax.ShapeDtypeStruct((n_edges, 16), jnp.float32),
            jax.ShapeDtypeStruct((n_edges // P, 4 * P), jnp.float32),
        ],
        mesh=plsc.VectorSubcoreMesh(core_axis_name="c", subcore_axis_name="s"),
        compiler_params=_SC_PARAMS,
        scratch_types=[
            pltpu.VMEM((8, 2, 128), jnp.int32),
            pltpu.VMEM((4096,), jnp.float32),
            pltpu.VMEM((1024, 8), jnp.float32),
            pltpu.VMEM((1024, 8), jnp.float32),
            pltpu.VMEM((128, 32), jnp.float32),
            pltpu.VMEM((STCH, 8), jnp.float32),
            pltpu.VMEM_SHARED((n_nodes, 8), jnp.float32),
            pltpu.SemaphoreType.DMA,
        ],
    )
    def gather_k(h_hbm, idx3_hbm, ea3_hbm, out_hbm, eap_hbm, idx_v, ea_v,
                 gs_v, gd_v, eap_v, zbuf, table, sem):
        cid = lax.axis_index("c")
        sid = lax.axis_index("s")

        def stage(ch):
            pltpu.sync_copy(h_hbm.at[pl.ds(ch * STCH, STCH)], zbuf)
            pltpu.sync_copy(zbuf, table.at[pl.ds(ch * STCH, STCH)])

        _stage_loop(sid, n_st, stage)
        plsc.subcore_barrier()

        w = sid * NC + cid
        ch0 = base_n * w + jnp.minimum(w, rem)
        lane = lax.iota(jnp.int32, 16)
        kmask = [(lane & 3) == k for k in range(4)]
        sel = [4 * h + (lane >> 2) for h in range(4)]

        def body(i, c):
            ch = ch0 + i
            pltpu.sync_copy(idx3_hbm.at[pl.ds(ch * 8, 8)], idx_v)
            pltpu.sync_copy(ea3_hbm.at[pl.ds(ch * 8 * 512, 8 * 512)], ea_v)
            descs = []
            for t in range(8):
                descs.append(pltpu.async_copy(
                    table.at[idx_v.at[t, 0]],
                    gs_v.at[pl.ds(t * 128, 128)], sem))
                descs.append(pltpu.async_copy(
                    table.at[idx_v.at[t, 1]],
                    gd_v.at[pl.ds(t * 128, 128)], sem))
            for t in range(8):
                for m in range(8):
                    a = [ea_v[pl.ds(t * 512 + k * 128 + 16 * m, 16)]
                         for k in range(4)]
                    for q in range(4):
                        v = jnp.zeros((16,), jnp.float32)
                        for k in range(4):
                            v = jnp.where(
                                kmask[k],
                                a[k].at[sel[q]].get(
                                    mode='promise_in_bounds'), v)
                        eap_v[16 * t + 2 * m + q // 2,
                              pl.ds(16 * (q % 2), 16)] = v
            for d in descs:
                d.wait()
            pltpu.sync_copy(gs_v,
                            out_hbm.at[pl.ds(ch * 1024, 1024), pl.ds(0, 8)])
            pltpu.sync_copy(gd_v,
                            out_hbm.at[pl.ds(ch * 1024, 1024), pl.ds(8, 8)])
            pltpu.sync_copy(eap_v, eap_hbm.at[pl.ds(ch * 128, 128)])
            return c

        lax.fori_loop(0, base_n + jnp.where(w < rem, 1, 0), body, 0)

    return gather_k


def _make_scatter(n_nodes, n_edges):
    n_ch = n_edges // 1024
    base_n = n_ch // NW
    rem = n_ch - base_n * NW
    n_st = n_nodes // STCH

    @functools.partial(
        pl.kernel,
        out_type=jax.ShapeDtypeStruct((NC, n_nodes, 8), jnp.float32),
        mesh=plsc.VectorSubcoreMesh(core_axis_name="c", subcore_axis_name="s"),
        compiler_params=_SC_PARAMS,
        scratch_types=[
            pltpu.VMEM((8, 2, 128), jnp.int32),
            pltpu.VMEM((1024, 8), jnp.float32),
            pltpu.VMEM((STCH, 8), jnp.float32),
            pltpu.VMEM_SHARED((n_nodes, 8), jnp.float32),
        ],
    )
    def scatter_k(enew_hbm, idx3_hbm, zeros_hbm, out_hbm, idx_v, ebuf, zbuf,
                  shared):
        cid = lax.axis_index("c")
        sid = lax.axis_index("s")

        def zstage(ch):
            pltpu.sync_copy(zeros_hbm.at[pl.ds(ch * STCH, STCH)], zbuf)
            pltpu.sync_copy(zbuf, shared.at[pl.ds(ch * STCH, STCH)])

        _stage_loop(sid, n_st, zstage)
        plsc.subcore_barrier()

        w = cid * NS + sid
        ch0 = base_n * w + jnp.minimum(w, rem)

        def outer(i, c):
            ch = ch0 + i
            pltpu.sync_copy(idx3_hbm.at[pl.ds(ch * 8, 8)], idx_v)
            pltpu.sync_copy(enew_hbm.at[pl.ds(ch * 1024, 1024)], ebuf)
            for t in range(8):
                pltpu.sync_copy(ebuf.at[pl.ds(t * 128, 128)],
                                shared.at[idx_v.at[t, 1]], add=True)
            return c

        lax.fori_loop(0, base_n + jnp.where(w < rem, 1, 0), outer, 0)
        plsc.subcore_barrier()

        def drain(ch):
            pltpu.sync_copy(shared.at[pl.ds(ch * STCH, STCH)], zbuf)
            pltpu.sync_copy(zbuf, out_hbm.at[cid, pl.ds(ch * STCH, STCH)])

        _stage_loop(sid, n_st, drain)

    return scatter_k


def kernel(x, edge_index, edge_attr, layer,
           W_ne1, W_ne2, W_ee1, W_ee2,
           W_r1, b_r1, W_r2, b_r2, W_o1, b_o1, W_o2, b_o2,
           W_b1, b_b1, W_b2, b_b2, W_b3, b_b3,
           W_c1, b_c1, W_c2, b_c2, W_c3, b_c3,
           latent_norm):
    f32 = jnp.float32
    N, D = x.shape
    E = edge_attr.shape[0]
    HID = W_ne1.shape[1]

    eye = jnp.eye(P, dtype=f32)
    w_ne2p = jnp.zeros((HID, 8), f32).at[:, :5].set(W_ne2)
    w1sd = (jnp.zeros((16, HID), f32)
            .at[0:5].set(W_r1[0:5]).at[8:13].set(W_r1[5:10]))
    wg = jnp.kron(eye, w1sd)
    we1 = jnp.kron(eye, W_ee1)
    we2 = jnp.kron(eye, W_ee2)
    wc = jnp.kron(eye, W_r1[10:14])
    wr2 = jnp.kron(eye, jnp.zeros((HID, 8), f32).at[:, :4].set(W_r2))
    br1 = jnp.tile(b_r1, P).reshape(1, P * HID)
    br2 = jnp.tile(jnp.zeros((8,), f32).at[:4].set(b_r2), P).reshape(1, 8 * P)

    wo1h = jnp.zeros((8, HID), f32).at[0:5].set(W_o1[0:5])
    wo1a = jnp.zeros((8, HID), f32).at[0:4].set(W_o1[5:9])
    bo1 = b_o1.reshape(1, HID)
    wo2 = jnp.zeros((HID, 8), f32).at[:, :5].set(W_o2)
    bo2 = jnp.zeros((1, 8), f32).at[0, :5].set(b_o2)
    wbc1 = jnp.zeros((8, 2 * HID), f32).at[0:5].set(
        jnp.concatenate([W_b1, W_c1], axis=1))
    bbc1 = jnp.concatenate([b_b1, b_c1]).reshape(1, 2 * HID)
    w2 = (jnp.zeros((2 * HID, 2 * HID), f32)
          .at[:HID, :HID].set(W_b2).at[HID:, HID:].set(W_c2))
    b2 = jnp.concatenate([b_b2, b_c2]).reshape(1, 2 * HID)
    w3 = (jnp.zeros((2 * HID, 8), f32)
          .at[:HID, 0:1].set(W_b3).at[HID:, 1:3].set(W_c3))
    b3 = jnp.concatenate([b_b3, b_c3, jnp.zeros((5,), f32)]).reshape(1, 8)
    ln = latent_norm.reshape(1, 1)

    idx3 = edge_index.reshape(2, E // 128, 128).transpose(1, 0, 2)
    ea3 = edge_attr.T.reshape(4, E // 128, 128).transpose(1, 0, 2)
    ea3f = ea3.reshape(4 * E)

    BN = 2000
    h8 = pl.pallas_call(
        _node_enc_body,
        grid=(N // BN,),
        in_specs=[
            pl.BlockSpec((BN, D), lambda i: (i, 0)),
            pl.BlockSpec((D, HID), lambda i: (0, 0)),
            pl.BlockSpec((HID, 8), lambda i: (0, 0)),
        ],
        out_specs=pl.BlockSpec((BN, 8), lambda i: (i, 0)),
        out_shape=jax.ShapeDtypeStruct((N, 8), f32),
    )(x, W_ne1, w_ne2p)

    g, eap = _make_gather(E, N)(h8, idx3, ea3f)

    R = E // P
    BR = 2000
    gp = g.reshape(R, 16 * P)
    enew = pl.pallas_call(
        _edge_body,
        grid=(R // BR,),
        in_specs=[
            pl.BlockSpec((BR, 16 * P), lambda i: (i, 0)),
            pl.BlockSpec((BR, 4 * P), lambda i: (i, 0)),
            pl.BlockSpec((16 * P, HID * P), lambda i: (0, 0)),
            pl.BlockSpec((4 * P, HID * P), lambda i: (0, 0)),
            pl.BlockSpec((HID * P, 4 * P), lambda i: (0, 0)),
            pl.BlockSpec((4 * P, HID * P), lambda i: (0, 0)),
            pl.BlockSpec((HID * P, 8 * P), lambda i: (0, 0)),
            pl.BlockSpec((1, HID * P), lambda i: (0, 0)),
            pl.BlockSpec((1, 8 * P), lambda i: (0, 0)),
        ],
        out_specs=pl.BlockSpec((BR, 8 * P), lambda i: (i, 0)),
        out_shape=jax.ShapeDtypeStruct((R, 8 * P), f32),
    )(gp, eap, wg, we1, we2, wc, wr2, br1, br2)

    enew8 = enew.reshape(E, 8)
    partials = _make_scatter(N, E)(enew8, idx3, jnp.zeros((N, 8), f32))

    out8 = pl.pallas_call(
        _node_out_body,
        grid=(N // BN,),
        in_specs=[
            pl.BlockSpec((BN, 8), lambda i: (i, 0)),
            pl.BlockSpec((BN, 8), lambda i: (i, 0)),
            pl.BlockSpec((BN, 8), lambda i: (i, 0)),
            pl.BlockSpec((8, HID), lambda i: (0, 0)),
            pl.BlockSpec((8, HID), lambda i: (0, 0)),
            pl.BlockSpec((1, HID), lambda i: (0, 0)),
            pl.BlockSpec((HID, 8), lambda i: (0, 0)),
            pl.BlockSpec((1, 8), lambda i: (0, 0)),
            pl.BlockSpec((8, 2 * HID), lambda i: (0, 0)),
            pl.BlockSpec((1, 2 * HID), lambda i: (0, 0)),
            pl.BlockSpec((2 * HID, 2 * HID), lambda i: (0, 0)),
            pl.BlockSpec((1, 2 * HID), lambda i: (0, 0)),
            pl.BlockSpec((2 * HID, 8), lambda i: (0, 0)),
            pl.BlockSpec((1, 8), lambda i: (0, 0)),
            pl.BlockSpec((1, 1), lambda i: (0, 0)),
        ],
        out_specs=pl.BlockSpec((BN, 8), lambda i: (i, 0)),
        out_shape=jax.ShapeDtypeStruct((N, 8), f32),
    )(h8, partials[0], partials[1], wo1h, wo1a, bo1, wo2, bo2,
      wbc1, bbc1, w2, b2, w3, b3, ln)

    return out8[:, 1:3], out8[:, 0]


# speedup vs baseline: 20.2174x; 1.1168x over previous
import functools

import jax
import jax.numpy as jnp
from jax import lax
from jax.experimental import pallas as pl
from jax.experimental.pallas import tpu as pltpu
from jax.experimental.pallas import tpu_sc as plsc

NC, NS = 2, 16
NW = NC * NS
P = 8
STCH = 2000

_relu = lambda v: jnp.maximum(v, 0.0)
_SC_PARAMS = pltpu.CompilerParams(use_tc_tiling_on_sc=False)
_SC_PARAMS_NL = pltpu.CompilerParams(use_tc_tiling_on_sc=False,
                                     needs_layout_passes=False)


def _dot(a, b):
    return jnp.dot(a, b, preferred_element_type=jnp.float32)


def _node_enc_body(x_ref, w1_ref, w2_ref, o_ref):
    t = _relu(_dot(x_ref[...], w1_ref[...]))
    o_ref[...] = _relu(_dot(t, w2_ref[...]))


def _edge_body(g_ref, ea_ref, wg_ref, we1_ref, we2_ref, wc_ref, wr2_ref,
               br1_ref, br2_ref, o_ref):
    t1 = _relu(_dot(ea_ref[...], we1_ref[...]))
    e = _relu(_dot(t1, we2_ref[...]))
    c = _dot(e, wc_ref[...])
    hid = _relu(_dot(g_ref[...], wg_ref[...]) + c + br1_ref[...])
    o_ref[...] = _dot(hid, wr2_ref[...]) + br2_ref[...]


def _node_out_body(h_ref, p0_ref, p1_ref, wo1h_ref, wo1a_ref, bo1_ref,
                   wo2_ref, bo2_ref, wbc1_ref, bbc1_ref, w2_ref, b2_ref,
                   w3_ref, b3_ref, ln_ref, o_ref):
    agg = p0_ref[...] + p1_ref[...]
    hid = _relu(_dot(h_ref[...], wo1h_ref[...]) + _dot(agg, wo1a_ref[...])
                + bo1_ref[...])
    hn = _dot(hid, wo2_ref[...]) + bo2_ref[...]
    t = _relu(_dot(hn, wbc1_ref[...]) + bbc1_ref[...])
    t2 = _relu(_dot(t, w2_ref[...]) + b2_ref[...])
    o3 = _dot(t2, w3_ref[...]) + b3_ref[...]
    eps = 1e-6
    beta = eps + (1.0 - 2.0 * eps) * jax.nn.sigmoid(o3[:, 0:1])
    hc = o3[:, 1:3] * ln_ref[0, 0]
    pad = jnp.zeros((o3.shape[0], 5), jnp.float32)
    o_ref[...] = jnp.concatenate([beta, hc, pad], axis=1)


def _stage_loop(sid, n_st, body_fn):
    per_tile = (n_st + NS - 1) // NS

    def body(i, c):
        ch = sid + NS * i

        @pl.when(ch < n_st)
        def _():
            body_fn(ch)

        return c

    lax.fori_loop(0, per_tile, body, 0)


def _make_gather(n_edges, n_nodes):
    n_ch = n_edges // 1024
    base_n = n_ch // NW
    rem = n_ch - base_n * NW
    n_st = n_nodes // STCH

    @functools.partial(
        pl.kernel,
        out_type=[
            jax.ShapeDtypeStruct((n_edges, 16), jnp.float32),
            jax.ShapeDtypeStruct((n_edges // P, 4 * P), jnp.float32),
        ],
        mesh=plsc.VectorSubcoreMesh(core_axis_name="c", subcore_axis_name="s"),
        compiler_params=_SC_PARAMS,
        scratch_types=[
            pltpu.VMEM((8, 2, 128), jnp.int32),
            pltpu.VMEM((4096,), jnp.float32),
            pltpu.VMEM((2, 1024, 8), jnp.float32),
            pltpu.VMEM((2, 1024, 8), jnp.float32),
            pltpu.VMEM((2, 128, 32), jnp.float32),
            pltpu.VMEM((STCH, 8), jnp.float32),
            pltpu.VMEM_SHARED((n_nodes, 8), jnp.float32),
            pltpu.SemaphoreType.DMA,
            pltpu.SemaphoreType.DMA,
            pltpu.SemaphoreType.DMA,
        ],
    )
    def gather_k(h_hbm, idx3_hbm, ea3_hbm, out_hbm, eap_hbm, idx_v, ea_v,
                 gs_v, gd_v, eap_v, zbuf, table, sem, wsem0, wsem1):
        cid = lax.axis_index("c")
        sid = lax.axis_index("s")

        def stage(ch):
            pltpu.sync_copy(h_hbm.at[pl.ds(ch * STCH, STCH)], zbuf)
            pltpu.sync_copy(zbuf, table.at[pl.ds(ch * STCH, STCH)])

        _stage_loop(sid, n_st, stage)
        plsc.subcore_barrier()

        w = sid * NC + cid
        nw_ = base_n + jnp.where(w < rem, 1, 0)
        ch0 = base_n * w + jnp.minimum(w, rem)
        lane = lax.iota(jnp.int32, 16)
        kmask = [(lane & 3) == k for k in range(4)]
        sel = [4 * h + (lane >> 2) for h in range(4)]
        wsems = (wsem0, wsem1)

        def drain(b):
            pltpu.make_async_copy(
                gs_v.at[b], out_hbm.at[pl.ds(0, 1024), pl.ds(0, 8)],
                wsems[b]).wait()
            pltpu.make_async_copy(
                gd_v.at[b], out_hbm.at[pl.ds(0, 1024), pl.ds(8, 8)],
                wsems[b]).wait()
            pltpu.make_async_copy(
                eap_v.at[b], eap_hbm.at[pl.ds(0, 128)], wsems[b]).wait()

        def process(ci, b):
            ch = ch0 + ci
            pltpu.sync_copy(idx3_hbm.at[pl.ds(ch * 8, 8)], idx_v)
            pltpu.sync_copy(ea3_hbm.at[pl.ds(ch * 4096, 4096)], ea_v)

            @pl.when(ci >= 2)
            def _():
                drain(b)

            descs = []
            for t in range(8):
                descs.append(pltpu.async_copy(
                    table.at[idx_v.at[t, 0]],
                    gs_v.at[b, pl.ds(t * 128, 128)], sem))
                descs.append(pltpu.async_copy(
                    table.at[idx_v.at[t, 1]],
                    gd_v.at[b, pl.ds(t * 128, 128)], sem))
            for t in range(8):
                for m in range(8):
                    a = [ea_v[pl.ds(t * 512 + k * 128 + 16 * m, 16)]
                         for k in range(4)]
                    for q in range(4):
                        v = jnp.zeros((16,), jnp.float32)
                        for k in range(4):
                            v = jnp.where(
                                kmask[k],
                                a[k].at[sel[q]].get(
                                    mode='promise_in_bounds'), v)
                        eap_v[b, 16 * t + 2 * m + q // 2,
                              pl.ds(16 * (q % 2), 16)] = v
            for d in descs:
                d.wait()
            pltpu.async_copy(gs_v.at[b],
                             out_hbm.at[pl.ds(ch * 1024, 1024), pl.ds(0, 8)],
                             wsems[b])
            pltpu.async_copy(gd_v.at[b],
                             out_hbm.at[pl.ds(ch * 1024, 1024), pl.ds(8, 8)],
                             wsems[b])
            pltpu.async_copy(eap_v.at[b], eap_hbm.at[pl.ds(ch * 128, 128)],
                             wsems[b])

        def body(i, c):
            for b in range(2):
                ci = 2 * i + b

                @pl.when(ci < nw_)
                def _(ci=ci, b=b):
                    process(ci, b)

            return c

        lax.fori_loop(0, (base_n + 2) // 2, body, 0)
        drain(0)
        drain(1)

    return gather_k


def _make_scatter(n_nodes, n_edges):
    n_ch = n_edges // 1024
    base_n = n_ch // NW
    rem = n_ch - base_n * NW
    n_st = n_nodes // STCH

    @functools.partial(
        pl.kernel,
        out_type=jax.ShapeDtypeStruct((NC, n_nodes, 8), jnp.float32),
        mesh=plsc.VectorSubcoreMesh(core_axis_name="c", subcore_axis_name="s"),
        compiler_params=_SC_PARAMS,
        scratch_types=[
            pltpu.VMEM((8, 2, 128), jnp.int32),
            pltpu.VMEM((1024, 8), jnp.float32),
            pltpu.VMEM((STCH, 8), jnp.float32),
            pltpu.VMEM_SHARED((n_nodes, 8), jnp.float32),
        ],
    )
    def scatter_k(enew_hbm, idx3_hbm, zeros_hbm, out_hbm, idx_v, ebuf, zbuf,
                  shared):
        cid = lax.axis_index("c")
        sid = lax.axis_index("s")

        def zstage(ch):
            pltpu.sync_copy(zeros_hbm.at[pl.ds(ch * STCH, STCH)], zbuf)
            pltpu.sync_copy(zbuf, shared.at[pl.ds(ch * STCH, STCH)])

        _stage_loop(sid, n_st, zstage)
        plsc.subcore_barrier()

        w = cid * NS + sid
        ch0 = base_n * w + jnp.minimum(w, rem)

        def outer(i, c):
            ch = ch0 + i
            pltpu.sync_copy(idx3_hbm.at[pl.ds(ch * 8, 8)], idx_v)
            pltpu.sync_copy(enew_hbm.at[pl.ds(ch * 1024, 1024)], ebuf)
            for t in range(8):
                pltpu.sync_copy(ebuf.at[pl.ds(t * 128, 128)],
                                shared.at[idx_v.at[t, 1]], add=True)
            return c

        lax.fori_loop(0, base_n + jnp.where(w < rem, 1, 0), outer, 0)
        plsc.subcore_barrier()

        def drain(ch):
            pltpu.sync_copy(shared.at[pl.ds(ch * STCH, STCH)], zbuf)
            pltpu.sync_copy(zbuf, out_hbm.at[cid, pl.ds(ch * STCH, STCH)])

        _stage_loop(sid, n_st, drain)

    return scatter_k


def kernel(x, edge_index, edge_attr, layer,
           W_ne1, W_ne2, W_ee1, W_ee2,
           W_r1, b_r1, W_r2, b_r2, W_o1, b_o1, W_o2, b_o2,
           W_b1, b_b1, W_b2, b_b2, W_b3, b_b3,
           W_c1, b_c1, W_c2, b_c2, W_c3, b_c3,
           latent_norm):
    f32 = jnp.float32
    N, D = x.shape
    E = edge_attr.shape[0]
    HID = W_ne1.shape[1]

    eye = jnp.eye(P, dtype=f32)
    w_ne2p = jnp.zeros((HID, 8), f32).at[:, :5].set(W_ne2)
    w1sd = (jnp.zeros((16, HID), f32)
            .at[0:5].set(W_r1[0:5]).at[8:13].set(W_r1[5:10]))
    wg = jnp.kron(eye, w1sd)
    we1 = jnp.kron(eye, W_ee1)
    we2 = jnp.kron(eye, W_ee2)
    wc = jnp.kron(eye, W_r1[10:14])
    wr2 = jnp.kron(eye, jnp.zeros((HID, 8), f32).at[:, :4].set(W_r2))
    br1 = jnp.tile(b_r1, P).reshape(1, P * HID)
    br2 = jnp.tile(jnp.zeros((8,), f32).at[:4].set(b_r2), P).reshape(1, 8 * P)

    wo1h = jnp.zeros((8, HID), f32).at[0:5].set(W_o1[0:5])
    wo1a = jnp.zeros((8, HID), f32).at[0:4].set(W_o1[5:9])
    bo1 = b_o1.reshape(1, HID)
    wo2 = jnp.zeros((HID, 8), f32).at[:, :5].set(W_o2)
    bo2 = jnp.zeros((1, 8), f32).at[0, :5].set(b_o2)
    wbc1 = jnp.zeros((8, 2 * HID), f32).at[0:5].set(
        jnp.concatenate([W_b1, W_c1], axis=1))
    bbc1 = jnp.concatenate([b_b1, b_c1]).reshape(1, 2 * HID)
    w2 = (jnp.zeros((2 * HID, 2 * HID), f32)
          .at[:HID, :HID].set(W_b2).at[HID:, HID:].set(W_c2))
    b2 = jnp.concatenate([b_b2, b_c2]).reshape(1, 2 * HID)
    w3 = (jnp.zeros((2 * HID, 8), f32)
          .at[:HID, 0:1].set(W_b3).at[HID:, 1:3].set(W_c3))
    b3 = jnp.concatenate([b_b3, b_c3, jnp.zeros((5,), f32)]).reshape(1, 8)
    ln = latent_norm.reshape(1, 1)

    idx3 = edge_index.reshape(2, E // 128, 128).transpose(1, 0, 2)
    ea3 = edge_attr.T.reshape(4, E // 128, 128).transpose(1, 0, 2)
    ea3f = ea3.reshape(4 * E)

    BN = 2000
    h8 = pl.pallas_call(
        _node_enc_body,
        grid=(N // BN,),
        in_specs=[
            pl.BlockSpec((BN, D), lambda i: (i, 0)),
            pl.BlockSpec((D, HID), lambda i: (0, 0)),
            pl.BlockSpec((HID, 8), lambda i: (0, 0)),
        ],
        out_specs=pl.BlockSpec((BN, 8), lambda i: (i, 0)),
        out_shape=jax.ShapeDtypeStruct((N, 8), f32),
    )(x, W_ne1, w_ne2p)

    g, eap = _make_gather(E, N)(h8, idx3, ea3f)

    R = E // P
    BR = 2000
    gp = g.reshape(R, 16 * P)
    enew = pl.pallas_call(
        _edge_body,
        grid=(R // BR,),
        in_specs=[
            pl.BlockSpec((BR, 16 * P), lambda i: (i, 0)),
            pl.BlockSpec((BR, 4 * P), lambda i: (i, 0)),
            pl.BlockSpec((16 * P, HID * P), lambda i: (0, 0)),
            pl.BlockSpec((4 * P, HID * P), lambda i: (0, 0)),
            pl.BlockSpec((HID * P, 4 * P), lambda i: (0, 0)),
            pl.BlockSpec((4 * P, HID * P), lambda i: (0, 0)),
            pl.BlockSpec((HID * P, 8 * P), lambda i: (0, 0)),
            pl.BlockSpec((1, HID * P), lambda i: (0, 0)),
            pl.BlockSpec((1, 8 * P), lambda i: (0, 0)),
        ],
        out_specs=pl.BlockSpec((BR, 8 * P), lambda i: (i, 0)),
        out_shape=jax.ShapeDtypeStruct((R, 8 * P), f32),
    )(gp, eap, wg, we1, we2, wc, wr2, br1, br2)

    enew8 = enew.reshape(E, 8)
    partials = _make_scatter(N, E)(enew8, idx3, jnp.zeros((N, 8), f32))

    out8 = pl.pallas_call(
        _node_out_body,
        grid=(N // BN,),
        in_specs=[
            pl.BlockSpec((BN, 8), lambda i: (i, 0)),
            pl.BlockSpec((BN, 8), lambda i: (i, 0)),
            pl.BlockSpec((BN, 8), lambda i: (i, 0)),
            pl.BlockSpec((8, HID), lambda i: (0, 0)),
            pl.BlockSpec((8, HID), lambda i: (0, 0)),
            pl.BlockSpec((1, HID), lambda i: (0, 0)),
            pl.BlockSpec((HID, 8), lambda i: (0, 0)),
            pl.BlockSpec((1, 8), lambda i: (0, 0)),
            pl.BlockSpec((8, 2 * HID), lambda i: (0, 0)),
            pl.BlockSpec((1, 2 * HID), lambda i: (0, 0)),
            pl.BlockSpec((2 * HID, 2 * HID), lambda i: (0, 0)),
            pl.BlockSpec((1, 2 * HID), lambda i: (0, 0)),
            pl.BlockSpec((2 * HID, 8), lambda i: (0, 0)),
            pl.BlockSpec((1, 8), lambda i: (0, 0)),
            pl.BlockSpec((1, 1), lambda i: (0, 0)),
        ],
        out_specs=pl.BlockSpec((BN, 8), lambda i: (i, 0)),
        out_shape=jax.ShapeDtypeStruct((N, 8), f32),
    )(h8, partials[0], partials[1], wo1h, wo1a, bo1, wo2, bo2,
      wbc1, bbc1, w2, b2, w3, b3, ln)

    return out8[:, 1:3], out8[:, 0]


# speedup vs baseline: 20.8036x; 1.0290x over previous
import functools

import jax
import jax.numpy as jnp
from jax import lax
from jax.experimental import pallas as pl
from jax.experimental.pallas import tpu as pltpu
from jax.experimental.pallas import tpu_sc as plsc

NC, NS = 2, 16
NW = NC * NS
P = 8
STCH = 2000

_relu = lambda v: jnp.maximum(v, 0.0)
_SC_PARAMS = pltpu.CompilerParams(use_tc_tiling_on_sc=False)
_SC_PARAMS_NL = pltpu.CompilerParams(use_tc_tiling_on_sc=False,
                                     needs_layout_passes=False)


def _dot(a, b):
    return jnp.dot(a, b, preferred_element_type=jnp.float32)


def _node_enc_body(x_ref, w1_ref, w2_ref, o_ref):
    t = _relu(_dot(x_ref[...], w1_ref[...]))
    o_ref[...] = _relu(_dot(t, w2_ref[...]))


def _edge_body(g_ref, ea_ref, wg_ref, we1_ref, we2_ref, wc_ref, wr2_ref,
               br1_ref, br2_ref, o_ref):
    t1 = _relu(_dot(ea_ref[...], we1_ref[...]))
    e = _relu(_dot(t1, we2_ref[...]))
    c = _dot(e, wc_ref[...])
    hid = _relu(_dot(g_ref[...], wg_ref[...]) + c + br1_ref[...])
    o_ref[...] = _dot(hid, wr2_ref[...]) + br2_ref[...]


def _node_out_body(h_ref, p0_ref, p1_ref, wo1h_ref, wo1a_ref, bo1_ref,
                   wo2_ref, bo2_ref, wbc1_ref, bbc1_ref, w2_ref, b2_ref,
                   w3_ref, b3_ref, ln_ref, o_ref):
    agg = p0_ref[...] + p1_ref[...]
    hid = _relu(_dot(h_ref[...], wo1h_ref[...]) + _dot(agg, wo1a_ref[...])
                + bo1_ref[...])
    hn = _dot(hid, wo2_ref[...]) + bo2_ref[...]
    t = _relu(_dot(hn, wbc1_ref[...]) + bbc1_ref[...])
    t2 = _relu(_dot(t, w2_ref[...]) + b2_ref[...])
    o3 = _dot(t2, w3_ref[...]) + b3_ref[...]
    eps = 1e-6
    beta = eps + (1.0 - 2.0 * eps) * jax.nn.sigmoid(o3[:, 0:1])
    hc = o3[:, 1:3] * ln_ref[0, 0]
    pad = jnp.zeros((o3.shape[0], 5), jnp.float32)
    o_ref[...] = jnp.concatenate([beta, hc, pad], axis=1)


def _stage_loop(sid, n_st, body_fn):
    per_tile = (n_st + NS - 1) // NS

    def body(i, c):
        ch = sid + NS * i

        @pl.when(ch < n_st)
        def _():
            body_fn(ch)

        return c

    lax.fori_loop(0, per_tile, body, 0)


def _make_gather(n_edges, n_nodes):
    n_ch = n_edges // 1024
    base_n = n_ch // NW
    rem = n_ch - base_n * NW
    n_st = n_nodes // STCH

    @functools.partial(
        pl.kernel,
        out_type=[
            jax.ShapeDtypeStruct((n_edges, 16), jnp.float32),
            jax.ShapeDtypeStruct((n_edges // P, 4 * P), jnp.float32),
        ],
        mesh=plsc.VectorSubcoreMesh(core_axis_name="c", subcore_axis_name="s"),
        compiler_params=_SC_PARAMS,
        scratch_types=[
            pltpu.VMEM((8, 2, 128), jnp.int32),
            pltpu.VMEM((4096,), jnp.float32),
            pltpu.VMEM((2, 1024, 8), jnp.float32),
            pltpu.VMEM((2, 1024, 8), jnp.float32),
            pltpu.VMEM((2, 128, 32), jnp.float32),
            pltpu.VMEM((STCH, 8), jnp.float32),
            pltpu.VMEM_SHARED((n_nodes, 8), jnp.float32),
            pltpu.SemaphoreType.DMA,
            pltpu.SemaphoreType.DMA,
            pltpu.SemaphoreType.DMA,
        ],
    )
    def gather_k(h_hbm, idx3_hbm, ea3_hbm, out_hbm, eap_hbm, idx_v, ea_v,
                 gs_v, gd_v, eap_v, zbuf, table, sem, wsem0, wsem1):
        cid = lax.axis_index("c")
        sid = lax.axis_index("s")

        def stage(ch):
            pltpu.sync_copy(h_hbm.at[pl.ds(ch * STCH, STCH)], zbuf)
            pltpu.sync_copy(zbuf, table.at[pl.ds(ch * STCH, STCH)])

        _stage_loop(sid, n_st, stage)
        plsc.subcore_barrier()

        w = sid * NC + cid
        nw_ = base_n + jnp.where(w < rem, 1, 0)
        ch0 = base_n * w + jnp.minimum(w, rem)
        lane = lax.iota(jnp.int32, 16)
        kmask = [(lane & 3) == k for k in range(4)]
        sel = [4 * h + (lane >> 2) for h in range(4)]
        wsems = (wsem0, wsem1)

        def drain(b):
            pltpu.make_async_copy(
                gs_v.at[b], out_hbm.at[pl.ds(0, 1024), pl.ds(0, 8)],
                wsems[b]).wait()
            pltpu.make_async_copy(
                gd_v.at[b], out_hbm.at[pl.ds(0, 1024), pl.ds(8, 8)],
                wsems[b]).wait()
            pltpu.make_async_copy(
                eap_v.at[b], eap_hbm.at[pl.ds(0, 128)], wsems[b]).wait()

        def process(ci, b):
            ch = ch0 + ci
            pltpu.sync_copy(idx3_hbm.at[pl.ds(ch * 8, 8)], idx_v)
            pltpu.sync_copy(ea3_hbm.at[pl.ds(ch * 4096, 4096)], ea_v)

            @pl.when(ci >= 2)
            def _():
                drain(b)

            descs = []
            for t in range(8):
                descs.append(pltpu.async_copy(
                    table.at[idx_v.at[t, 0]],
                    gs_v.at[b, pl.ds(t * 128, 128)], sem))
                descs.append(pltpu.async_copy(
                    table.at[idx_v.at[t, 1]],
                    gd_v.at[b, pl.ds(t * 128, 128)], sem))
            for t in range(8):
                for m in range(8):
                    a = [ea_v[pl.ds(t * 512 + k * 128 + 16 * m, 16)]
                         for k in range(4)]
                    for q in range(4):
                        v = jnp.zeros((16,), jnp.float32)
                        for k in range(4):
                            v = jnp.where(
                                kmask[k],
                                a[k].at[sel[q]].get(
                                    mode='promise_in_bounds'), v)
                        eap_v[b, 16 * t + 2 * m + q // 2,
                              pl.ds(16 * (q % 2), 16)] = v
            for d in descs:
                d.wait()
            pltpu.async_copy(gs_v.at[b],
                             out_hbm.at[pl.ds(ch * 1024, 1024), pl.ds(0, 8)],
                             wsems[b])
            pltpu.async_copy(gd_v.at[b],
                             out_hbm.at[pl.ds(ch * 1024, 1024), pl.ds(8, 8)],
                             wsems[b])
            pltpu.async_copy(eap_v.at[b], eap_hbm.at[pl.ds(ch * 128, 128)],
                             wsems[b])

        def body(i, c):
            for b in range(2):
                ci = 2 * i + b

                @pl.when(ci < nw_)
                def _(ci=ci, b=b):
                    process(ci, b)

            return c

        lax.fori_loop(0, (base_n + 2) // 2, body, 0)
        drain(0)
        drain(1)

    return gather_k


def _make_scatter(n_nodes, n_edges):
    n_ch = n_edges // 1024
    base_n = n_ch // NW
    rem = n_ch - base_n * NW
    n_st = n_nodes // STCH

    @functools.partial(
        pl.kernel,
        out_type=jax.ShapeDtypeStruct((NC, n_nodes, 8), jnp.float32),
        mesh=plsc.VectorSubcoreMesh(core_axis_name="c", subcore_axis_name="s"),
        compiler_params=_SC_PARAMS,
        scratch_types=[
            pltpu.VMEM((8, 2, 128), jnp.int32),
            pltpu.VMEM((1024, 8), jnp.float32),
            pltpu.VMEM((STCH, 8), jnp.float32),
            pltpu.VMEM_SHARED((n_nodes, 8), jnp.float32),
            pltpu.SemaphoreType.DMA,
        ],
    )
    def scatter_k(enew_hbm, idx3_hbm, zeros_hbm, out_hbm, idx_v, ebuf, zbuf,
                  shared, ssem):
        cid = lax.axis_index("c")
        sid = lax.axis_index("s")

        def zstage(ch):
            pltpu.sync_copy(zeros_hbm.at[pl.ds(ch * STCH, STCH)], zbuf)
            pltpu.sync_copy(zbuf, shared.at[pl.ds(ch * STCH, STCH)])

        _stage_loop(sid, n_st, zstage)
        plsc.subcore_barrier()

        w = cid * NS + sid
        ch0 = base_n * w + jnp.minimum(w, rem)

        def outer(i, c):
            ch = ch0 + i
            pltpu.sync_copy(idx3_hbm.at[pl.ds(ch * 8, 8)], idx_v)
            pltpu.sync_copy(enew_hbm.at[pl.ds(ch * 1024, 1024)], ebuf)
            descs = [pltpu.async_copy(ebuf.at[pl.ds(t * 128, 128)],
                                      shared.at[idx_v.at[t, 1]], ssem,
                                      add=True)
                     for t in range(8)]
            for d in descs:
                d.wait()
            return c

        lax.fori_loop(0, base_n + jnp.where(w < rem, 1, 0), outer, 0)
        plsc.subcore_barrier()

        def drain(ch):
            pltpu.sync_copy(shared.at[pl.ds(ch * STCH, STCH)], zbuf)
            pltpu.sync_copy(zbuf, out_hbm.at[cid, pl.ds(ch * STCH, STCH)])

        _stage_loop(sid, n_st, drain)

    return scatter_k


def kernel(x, edge_index, edge_attr, layer,
           W_ne1, W_ne2, W_ee1, W_ee2,
           W_r1, b_r1, W_r2, b_r2, W_o1, b_o1, W_o2, b_o2,
           W_b1, b_b1, W_b2, b_b2, W_b3, b_b3,
           W_c1, b_c1, W_c2, b_c2, W_c3, b_c3,
           latent_norm):
    f32 = jnp.float32
    N, D = x.shape
    E = edge_attr.shape[0]
    HID = W_ne1.shape[1]

    eye = jnp.eye(P, dtype=f32)
    w_ne2p = jnp.zeros((HID, 8), f32).at[:, :5].set(W_ne2)
    w1sd = (jnp.zeros((16, HID), f32)
            .at[0:5].set(W_r1[0:5]).at[8:13].set(W_r1[5:10]))
    wg = jnp.kron(eye, w1sd)
    we1 = jnp.kron(eye, W_ee1)
    we2 = jnp.kron(eye, W_ee2)
    wc = jnp.kron(eye, W_r1[10:14])
    wr2 = jnp.kron(eye, jnp.zeros((HID, 8), f32).at[:, :4].set(W_r2))
    br1 = jnp.tile(b_r1, P).reshape(1, P * HID)
    br2 = jnp.tile(jnp.zeros((8,), f32).at[:4].set(b_r2), P).reshape(1, 8 * P)

    wo1h = jnp.zeros((8, HID), f32).at[0:5].set(W_o1[0:5])
    wo1a = jnp.zeros((8, HID), f32).at[0:4].set(W_o1[5:9])
    bo1 = b_o1.reshape(1, HID)
    wo2 = jnp.zeros((HID, 8), f32).at[:, :5].set(W_o2)
    bo2 = jnp.zeros((1, 8), f32).at[0, :5].set(b_o2)
    wbc1 = jnp.zeros((8, 2 * HID), f32).at[0:5].set(
        jnp.concatenate([W_b1, W_c1], axis=1))
    bbc1 = jnp.concatenate([b_b1, b_c1]).reshape(1, 2 * HID)
    w2 = (jnp.zeros((2 * HID, 2 * HID), f32)
          .at[:HID, :HID].set(W_b2).at[HID:, HID:].set(W_c2))
    b2 = jnp.concatenate([b_b2, b_c2]).reshape(1, 2 * HID)
    w3 = (jnp.zeros((2 * HID, 8), f32)
          .at[:HID, 0:1].set(W_b3).at[HID:, 1:3].set(W_c3))
    b3 = jnp.concatenate([b_b3, b_c3, jnp.zeros((5,), f32)]).reshape(1, 8)
    ln = latent_norm.reshape(1, 1)

    idx3 = edge_index.reshape(2, E // 128, 128).transpose(1, 0, 2)
    ea3 = edge_attr.T.reshape(4, E // 128, 128).transpose(1, 0, 2)
    ea3f = ea3.reshape(4 * E)

    BN = 2000
    h8 = pl.pallas_call(
        _node_enc_body,
        grid=(N // BN,),
        in_specs=[
            pl.BlockSpec((BN, D), lambda i: (i, 0)),
            pl.BlockSpec((D, HID), lambda i: (0, 0)),
            pl.BlockSpec((HID, 8), lambda i: (0, 0)),
        ],
        out_specs=pl.BlockSpec((BN, 8), lambda i: (i, 0)),
        out_shape=jax.ShapeDtypeStruct((N, 8), f32),
    )(x, W_ne1, w_ne2p)

    g, eap = _make_gather(E, N)(h8, idx3, ea3f)

    R = E // P
    BR = 2000
    gp = g.reshape(R, 16 * P)
    enew = pl.pallas_call(
        _edge_body,
        grid=(R // BR,),
        in_specs=[
            pl.BlockSpec((BR, 16 * P), lambda i: (i, 0)),
            pl.BlockSpec((BR, 4 * P), lambda i: (i, 0)),
            pl.BlockSpec((16 * P, HID * P), lambda i: (0, 0)),
            pl.BlockSpec((4 * P, HID * P), lambda i: (0, 0)),
            pl.BlockSpec((HID * P, 4 * P), lambda i: (0, 0)),
            pl.BlockSpec((4 * P, HID * P), lambda i: (0, 0)),
            pl.BlockSpec((HID * P, 8 * P), lambda i: (0, 0)),
            pl.BlockSpec((1, HID * P), lambda i: (0, 0)),
            pl.BlockSpec((1, 8 * P), lambda i: (0, 0)),
        ],
        out_specs=pl.BlockSpec((BR, 8 * P), lambda i: (i, 0)),
        out_shape=jax.ShapeDtypeStruct((R, 8 * P), f32),
    )(gp, eap, wg, we1, we2, wc, wr2, br1, br2)

    enew8 = enew.reshape(E, 8)
    partials = _make_scatter(N, E)(enew8, idx3, jnp.zeros((N, 8), f32))

    out8 = pl.pallas_call(
        _node_out_body,
        grid=(N // BN,),
        in_specs=[
            pl.BlockSpec((BN, 8), lambda i: (i, 0)),
            pl.BlockSpec((BN, 8), lambda i: (i, 0)),
            pl.BlockSpec((BN, 8), lambda i: (i, 0)),
            pl.BlockSpec((8, HID), lambda i: (0, 0)),
            pl.BlockSpec((8, HID), lambda i: (0, 0)),
            pl.BlockSpec((1, HID), lambda i: (0, 0)),
            pl.BlockSpec((HID, 8), lambda i: (0, 0)),
            pl.BlockSpec((1, 8), lambda i: (0, 0)),
            pl.BlockSpec((8, 2 * HID), lambda i: (0, 0)),
            pl.BlockSpec((1, 2 * HID), lambda i: (0, 0)),
            pl.BlockSpec((2 * HID, 2 * HID), lambda i: (0, 0)),
            pl.BlockSpec((1, 2 * HID), lambda i: (0, 0)),
            pl.BlockSpec((2 * HID, 8), lambda i: (0, 0)),
            pl.BlockSpec((1, 8), lambda i: (0, 0)),
            pl.BlockSpec((1, 1), lambda i: (0, 0)),
        ],
        out_specs=pl.BlockSpec((BN, 8), lambda i: (i, 0)),
        out_shape=jax.ShapeDtypeStruct((N, 8), f32),
    )(h8, partials[0], partials[1], wo1h, wo1a, bo1, wo2, bo2,
      wbc1, bbc1, w2, b2, w3, b3, ln)

    return out8[:, 1:3], out8[:, 0]
